# Initial kernel scaffold; baseline (speedup 1.0000x reference)
import jax
import jax.numpy as jnp
from jax.experimental import pallas as pl


def kernel(x_cat, x_cont, y_cat, y_cont, bandwidth, sqrt_scale, std, covar_factor):
    raise NotImplementedError("write your pallas kernel here")


# speedup vs baseline: 3.9942x; 3.9942x over previous
import functools

import jax
import jax.numpy as jnp
from jax import lax
from jax.experimental import pallas as pl
from jax.experimental.pallas import tpu as pltpu
from jax.experimental.pallas import tpu_sc as plsc


def _sc_cat_partials(tbl, stdp, x_t, y_t, B, F_CAT, NCAT, RANK, NPAD):
    info = plsc.get_sparse_core_info()
    NC, NS, L = info.num_cores, info.num_subcores, info.num_lanes
    groups = B // L

    @functools.partial(
        pl.kernel,
        out_type=jax.ShapeDtypeStruct((F_CAT, B), jnp.float32),
        mesh=plsc.VectorSubcoreMesh(core_axis_name="c", subcore_axis_name="s"),
        compiler_params=pltpu.CompilerParams(needs_layout_passes=False),
        scratch_types=[
            pltpu.VMEM((NCAT * RANK,), jnp.float32),
            pltpu.VMEM((NPAD,), jnp.float32),
            pltpu.VMEM((B,), jnp.int32),
            pltpu.VMEM((B,), jnp.int32),
            pltpu.VMEM((B,), jnp.float32),
        ],
    )
    def k(tbl_hbm, std_hbm, x_hbm, y_hbm, out_hbm, tbl_v, std_v, x_v, y_v, out_v):
        c = lax.axis_index("c")
        s = lax.axis_index("s")
        wid = s * NC + c

        @pl.when(wid < F_CAT)
        def _():
            pltpu.sync_copy(tbl_hbm.at[wid], tbl_v)
            pltpu.sync_copy(std_hbm.at[wid], std_v)
            pltpu.sync_copy(x_hbm.at[wid], x_v)
            pltpu.sync_copy(y_hbm.at[wid], y_v)

            def group(g, carry):
                base = g * L
                bx = x_v[pl.ds(base, L)]
                by = y_v[pl.ds(base, L)]
                ax = bx * RANK
                ay = by * RANK
                acc = jnp.zeros((L,), jnp.float32)
                for r in range(RANK):
                    vx = plsc.load_gather(tbl_v, [ax + r])
                    vy = plsc.load_gather(tbl_v, [ay + r])
                    acc = acc + vx * vy
                sv = plsc.load_gather(std_v, [bx])
                acc = acc + jnp.where(bx == by, sv * sv, jnp.zeros((L,), jnp.float32))
                out_v[pl.ds(base, L)] = acc
                return carry

            lax.fori_loop(0, groups, group, 0)
            pltpu.sync_copy(out_v, out_hbm.at[wid])

    return k(tbl, stdp, x_t, y_t)


def _tc_combine(partials, xc_t, yc_t, bw, ss, B, F_CAT, F_CONT):
    BLK = 2048

    def body(p_ref, x_ref, y_ref, bw_ref, ss_ref, o_ref):
        d = x_ref[...] - y_ref[...]
        bwv = bw_ref[...]
        inv = 1.0 / (2.0 * bwv * bwv)
        scale = ss_ref[...] * ss_ref[...]
        cont = jnp.sum(scale * jnp.exp(-(d * d) * inv), axis=0, keepdims=True)
        cat = jnp.sum(p_ref[...], axis=0, keepdims=True)
        o_ref[...] = cont + cat

    out = pl.pallas_call(
        body,
        grid=(B // BLK,),
        in_specs=[
            pl.BlockSpec((F_CAT, BLK), lambda i: (0, i)),
            pl.BlockSpec((F_CONT, BLK), lambda i: (0, i)),
            pl.BlockSpec((F_CONT, BLK), lambda i: (0, i)),
            pl.BlockSpec((F_CONT, 1), lambda i: (0, 0)),
            pl.BlockSpec((F_CONT, 1), lambda i: (0, 0)),
        ],
        out_specs=pl.BlockSpec((1, BLK), lambda i: (0, i)),
        out_shape=jax.ShapeDtypeStruct((1, B), jnp.float32),
    )(partials, xc_t, yc_t, bw, ss)
    return out.reshape(B)


def kernel(x_cat, x_cont, y_cat, y_cont, bandwidth, sqrt_scale, std, covar_factor):
    B, F_CAT = x_cat.shape
    F_CONT = x_cont.shape[1]
    NCAT = std.shape[1]
    RANK = covar_factor.shape[2]
    NPAD = NCAT + (-NCAT) % 256
    tbl = covar_factor.reshape(F_CAT, NCAT * RANK)
    stdp = jnp.pad(std, ((0, 0), (0, NPAD - NCAT)))
    partials = _sc_cat_partials(
        tbl, stdp, x_cat.T, y_cat.T, B, F_CAT, NCAT, RANK, NPAD
    )
    return _tc_combine(
        partials,
        x_cont.T,
        y_cont.T,
        bandwidth.reshape(F_CONT, 1),
        sqrt_scale.reshape(F_CONT, 1),
        B,
        F_CAT,
        F_CONT,
    )


# speedup vs baseline: 9.8237x; 2.4595x over previous
import functools

import jax
import jax.numpy as jnp
from jax import lax
from jax.experimental import pallas as pl
from jax.experimental.pallas import tpu as pltpu
from jax.experimental.pallas import tpu_sc as plsc


def _sc_cat_partials(tbl, stdp, x_t, y_t, B, F_CAT, NCAT, RANK, NPAD):
    info = plsc.get_sparse_core_info()
    NC, NS, L = info.num_cores, info.num_subcores, info.num_lanes
    groups = B // L

    @functools.partial(
        pl.kernel,
        out_type=jax.ShapeDtypeStruct((F_CAT, B), jnp.float32),
        mesh=plsc.VectorSubcoreMesh(core_axis_name="c", subcore_axis_name="s"),
        compiler_params=pltpu.CompilerParams(needs_layout_passes=False),
        scratch_types=[
            pltpu.VMEM((NPAD * RANK,), jnp.float32),
            pltpu.VMEM((NPAD,), jnp.float32),
            pltpu.VMEM((B,), jnp.int32),
            pltpu.VMEM((B,), jnp.int32),
            pltpu.VMEM((B,), jnp.float32),
        ],
    )
    def k(tbl_hbm, std_hbm, x_hbm, y_hbm, out_hbm, tbl_v, std_v, x_v, y_v, out_v):
        c = lax.axis_index("c")
        s = lax.axis_index("s")
        wid = s * NC + c

        @pl.when(wid < F_CAT)
        def _():
            pltpu.sync_copy(tbl_hbm.at[wid], tbl_v)
            pltpu.sync_copy(std_hbm.at[wid], std_v)
            pltpu.sync_copy(x_hbm.at[wid], x_v)
            pltpu.sync_copy(y_hbm.at[wid], y_v)

            def group(g, carry):
                base = g * L
                bx = x_v[pl.ds(base, L)]
                by = y_v[pl.ds(base, L)]
                acc = jnp.zeros((L,), jnp.float32)
                for r in range(RANK):
                    vx = plsc.load_gather(tbl_v, [bx + r * NPAD])
                    vy = plsc.load_gather(tbl_v, [by + r * NPAD])
                    acc = acc + vx * vy
                sv = plsc.load_gather(std_v, [bx])
                acc = acc + jnp.where(bx == by, sv * sv, jnp.zeros((L,), jnp.float32))
                out_v[pl.ds(base, L)] = acc
                return carry

            lax.fori_loop(0, groups, group, 0)
            pltpu.sync_copy(out_v, out_hbm.at[wid])

    return k(tbl, stdp, x_t, y_t)


def _tc_combine(partials, xc_t, yc_t, bw, ss, B, F_CAT, F_CONT):
    BLK = 2048

    def body(p_ref, x_ref, y_ref, bw_ref, ss_ref, o_ref):
        d = x_ref[...] - y_ref[...]
        bwv = bw_ref[...]
        inv = 1.0 / (2.0 * bwv * bwv)
        scale = ss_ref[...] * ss_ref[...]
        cont = jnp.sum(scale * jnp.exp(-(d * d) * inv), axis=0, keepdims=True)
        cat = jnp.sum(p_ref[...], axis=0, keepdims=True)
        o_ref[...] = cont + cat

    out = pl.pallas_call(
        body,
        grid=(B // BLK,),
        in_specs=[
            pl.BlockSpec((F_CAT, BLK), lambda i: (0, i)),
            pl.BlockSpec((F_CONT, BLK), lambda i: (0, i)),
            pl.BlockSpec((F_CONT, BLK), lambda i: (0, i)),
            pl.BlockSpec((F_CONT, 1), lambda i: (0, 0)),
            pl.BlockSpec((F_CONT, 1), lambda i: (0, 0)),
        ],
        out_specs=pl.BlockSpec((1, BLK), lambda i: (0, i)),
        out_shape=jax.ShapeDtypeStruct((1, B), jnp.float32),
    )(partials, xc_t, yc_t, bw, ss)
    return out.reshape(B)


def kernel(x_cat, x_cont, y_cat, y_cont, bandwidth, sqrt_scale, std, covar_factor):
    B, F_CAT = x_cat.shape
    F_CONT = x_cont.shape[1]
    NCAT = std.shape[1]
    RANK = covar_factor.shape[2]
    NPAD = NCAT + (-NCAT) % 256
    tbl = jnp.pad(
        covar_factor.transpose(0, 2, 1), ((0, 0), (0, 0), (0, NPAD - NCAT))
    ).reshape(F_CAT, RANK * NPAD)
    stdp = jnp.pad(std, ((0, 0), (0, NPAD - NCAT)))
    partials = _sc_cat_partials(
        tbl, stdp, x_cat.T, y_cat.T, B, F_CAT, NCAT, RANK, NPAD
    )
    return _tc_combine(
        partials,
        x_cont.T,
        y_cont.T,
        bandwidth.reshape(F_CONT, 1),
        sqrt_scale.reshape(F_CONT, 1),
        B,
        F_CAT,
        F_CONT,
    )


# speedup vs baseline: 9.8800x; 1.0057x over previous
import functools

import jax
import jax.numpy as jnp
from jax import lax
from jax.experimental import pallas as pl
from jax.experimental.pallas import tpu as pltpu
from jax.experimental.pallas import tpu_sc as plsc


def _sc_cat_partials(tbl, stdp, x_t, y_t, B, F_CAT, NCAT, RANK, NPAD):
    info = plsc.get_sparse_core_info()
    NC, NS, L = info.num_cores, info.num_subcores, info.num_lanes
    NW = NC * NS
    NPAIR = B * F_CAT
    P = NPAIR // NW
    FS = NPAD * RANK
    groups = P // L

    @functools.partial(
        pl.kernel,
        out_type=jax.ShapeDtypeStruct((NPAIR,), jnp.float32),
        mesh=plsc.VectorSubcoreMesh(core_axis_name="c", subcore_axis_name="s"),
        compiler_params=pltpu.CompilerParams(
            needs_layout_passes=False, use_tc_tiling_on_sc=False
        ),
        scratch_types=[
            pltpu.VMEM((2 * FS,), jnp.float32),
            pltpu.VMEM((2 * NPAD,), jnp.float32),
            pltpu.VMEM((P,), jnp.int32),
            pltpu.VMEM((P,), jnp.int32),
            pltpu.VMEM((P,), jnp.float32),
        ],
    )
    def k(tbl_hbm, std_hbm, x_hbm, y_hbm, out_hbm, tbl_v, std_v, x_v, y_v, out_v):
        c = lax.axis_index("c")
        s = lax.axis_index("s")
        wid = s * NC + c
        q0 = wid * P
        f0 = q0 // B
        pltpu.sync_copy(tbl_hbm.at[pl.ds(f0 * FS, 2 * FS)], tbl_v)
        pltpu.sync_copy(std_hbm.at[pl.ds(f0 * NPAD, 2 * NPAD)], std_v)
        pltpu.sync_copy(x_hbm.at[pl.ds(q0, P)], x_v)
        pltpu.sync_copy(y_hbm.at[pl.ds(q0, P)], y_v)

        def group(g, carry):
            base = g * L
            df = (q0 + base) // B - f0
            toff = jnp.full((L,), df * FS, jnp.int32)
            soff = jnp.full((L,), df * NPAD, jnp.int32)
            bx = x_v[pl.ds(base, L)]
            by = y_v[pl.ds(base, L)]
            bxt = bx | toff
            byt = by | toff
            acc = jnp.zeros((L,), jnp.float32)
            for r in range(RANK):
                vx = plsc.load_gather(tbl_v, [bxt | (r * NPAD)])
                vy = plsc.load_gather(tbl_v, [byt | (r * NPAD)])
                acc = acc + vx * vy
            sv = plsc.load_gather(std_v, [bx | soff])
            acc = acc + jnp.where(bx == by, sv * sv, jnp.zeros((L,), jnp.float32))
            out_v[pl.ds(base, L)] = acc
            return carry

        lax.fori_loop(0, groups, group, 0)
        pltpu.sync_copy(out_v, out_hbm.at[pl.ds(q0, P)])

    return k(tbl, stdp, x_t, y_t).reshape(F_CAT, B)


def _tc_combine(partials, xc_t, yc_t, bw, ss, B, F_CAT, F_CONT):
    BLK = 2048

    def body(p_ref, x_ref, y_ref, bw_ref, ss_ref, o_ref):
        d = x_ref[...] - y_ref[...]
        bwv = bw_ref[...]
        inv = 1.0 / (2.0 * bwv * bwv)
        scale = ss_ref[...] * ss_ref[...]
        cont = jnp.sum(scale * jnp.exp(-(d * d) * inv), axis=0, keepdims=True)
        cat = jnp.sum(p_ref[...], axis=0, keepdims=True)
        o_ref[...] = cont + cat

    out = pl.pallas_call(
        body,
        grid=(B // BLK,),
        in_specs=[
            pl.BlockSpec((F_CAT, BLK), lambda i: (0, i)),
            pl.BlockSpec((F_CONT, BLK), lambda i: (0, i)),
            pl.BlockSpec((F_CONT, BLK), lambda i: (0, i)),
            pl.BlockSpec((F_CONT, 1), lambda i: (0, 0)),
            pl.BlockSpec((F_CONT, 1), lambda i: (0, 0)),
        ],
        out_specs=pl.BlockSpec((1, BLK), lambda i: (0, i)),
        out_shape=jax.ShapeDtypeStruct((1, B), jnp.float32),
    )(partials, xc_t, yc_t, bw, ss)
    return out.reshape(B)


def kernel(x_cat, x_cont, y_cat, y_cont, bandwidth, sqrt_scale, std, covar_factor):
    B, F_CAT = x_cat.shape
    F_CONT = x_cont.shape[1]
    NCAT = std.shape[1]
    RANK = covar_factor.shape[2]
    NPAD = NCAT + (-NCAT) % 1024
    tbl = jnp.pad(
        covar_factor.transpose(0, 2, 1),
        ((0, 1), (0, 0), (0, NPAD - NCAT)),
    ).reshape(-1)
    stdp = jnp.pad(std, ((0, 1), (0, NPAD - NCAT))).reshape(-1)
    partials = _sc_cat_partials(
        tbl,
        stdp,
        x_cat.T.reshape(-1),
        y_cat.T.reshape(-1),
        B,
        F_CAT,
        NCAT,
        RANK,
        NPAD,
    )
    return _tc_combine(
        partials,
        x_cont.T,
        y_cont.T,
        bandwidth.reshape(F_CONT, 1),
        sqrt_scale.reshape(F_CONT, 1),
        B,
        F_CAT,
        F_CONT,
    )
